# baseline (speedup 1.0000x reference)
import functools

import jax
import jax.numpy as jnp
from jax import lax
from jax.experimental import pallas as pl
from jax.experimental.pallas import tpu as pltpu
from jax.experimental.pallas import tpu_sc as plsc

F = 128

NC = 2
NS = 16
NW = NC * NS
B = 128
NB = 79
NE_PAD = NW * NB * B
NBH = 40
NE_PAD2 = 2 * NW * NBH * B
RB = 1024
N_PAD = 10240
NBLK = N_PAD // RB
RPT = N_PAD // NS

_mesh = plsc.VectorSubcoreMesh(core_axis_name="c", subcore_axis_name="s")


@functools.partial(
    pl.kernel,
    mesh=_mesh,
    out_type=jax.ShapeDtypeStruct((NC, N_PAD), jnp.float32),
    scratch_types=[
        pltpu.VMEM((NB, B), jnp.int32),
        pltpu.VMEM((B,), jnp.float32),
        pltpu.VMEM_SHARED((N_PAD,), jnp.float32),
    ],
)
def _deg_kernel(dst_hbm, zeros_hbm, out_hbm, dst_v, ones_v, acc):
    c = lax.axis_index("c")
    s = lax.axis_index("s")
    wid = s * NC + c
    for i in range(B // 16):
        ones_v[pl.ds(i * 16, 16)] = jnp.ones((16,), jnp.float32)
    pltpu.sync_copy(zeros_hbm.at[pl.ds(s * RPT, RPT)], acc.at[pl.ds(s * RPT, RPT)])
    pltpu.sync_copy(dst_hbm.at[wid], dst_v)
    plsc.subcore_barrier()

    def body(j, carry):
        pltpu.sync_copy(ones_v, acc.at[dst_v.at[j]], add=True)
        return carry

    lax.fori_loop(0, NB, body, 0)
    plsc.subcore_barrier()
    pltpu.sync_copy(acc.at[pl.ds(s * RPT, RPT)], out_hbm.at[c, pl.ds(s * RPT, RPT)])


@functools.partial(
    pl.kernel,
    mesh=_mesh,
    out_type=jax.ShapeDtypeStruct((NC, N_PAD, F), jnp.float32),
    scratch_types=[
        pltpu.VMEM((NBH, B), jnp.int32),
        pltpu.VMEM((NBH, B), jnp.int32),
        pltpu.VMEM((2 * B, F), jnp.float32),
        pltpu.VMEM_SHARED((N_PAD, F), jnp.float32),
        pltpu.SemaphoreType.DMA,
    ],
)
def _agg_kernel(y_hbm, src_hbm, dst_hbm, init_hbm, out_hbm,
                src_v, dst_v, rows, acc, sem0):
    c = lax.axis_index("c")
    s = lax.axis_index("s")
    wid = s * NC + c
    rows0 = rows.at[pl.ds(0, B)]
    rows1 = rows.at[pl.ds(B, B)]
    pltpu.sync_copy(init_hbm.at[c, pl.ds(s * RPT, RPT)],
                    acc.at[pl.ds(s * RPT, RPT)])
    pltpu.sync_copy(src_hbm.at[wid], src_v)
    pltpu.sync_copy(dst_hbm.at[wid], dst_v)
    plsc.subcore_barrier()

    pltpu.async_copy(y_hbm.at[src_v.at[0]], rows0, sem0)
    pltpu.async_copy(y_hbm.at[src_v.at[1]], rows1, sem0)

    def body(i, carry):
        j = i * 2
        pltpu.make_async_copy(y_hbm.at[src_v.at[j]], rows0, sem0).wait()
        pltpu.sync_copy(rows0, acc.at[dst_v.at[j]], add=True)
        pltpu.async_copy(y_hbm.at[src_v.at[jnp.minimum(j + 2, NBH - 1)]],
                         rows0, sem0)
        pltpu.make_async_copy(y_hbm.at[src_v.at[j + 1]], rows1, sem0).wait()
        pltpu.sync_copy(rows1, acc.at[dst_v.at[j + 1]], add=True)
        pltpu.async_copy(y_hbm.at[src_v.at[jnp.minimum(j + 3, NBH - 1)]],
                         rows1, sem0)
        return carry

    lax.fori_loop(0, NBH // 2, body, 0)
    pltpu.make_async_copy(y_hbm.at[src_v.at[NBH - 1]], rows0, sem0).wait()
    pltpu.make_async_copy(y_hbm.at[src_v.at[NBH - 1]], rows1, sem0).wait()
    plsc.subcore_barrier()
    pltpu.sync_copy(acc.at[pl.ds(s * RPT, RPT)], out_hbm.at[c, pl.ds(s * RPT, RPT)])


def _dis(degp_blk):
    deg = degp_blk[:, 0:1] + degp_blk[:, 1:2] + 1.0
    return lax.rsqrt(deg)


def _k_scale_mm(x_ref, w_ref, degp_ref, y_ref):
    xw = jnp.dot(x_ref[...], w_ref[...], preferred_element_type=jnp.float32)
    y_ref[...] = xw * _dis(degp_ref[...])


def _k_layer(p_ref, y1_ref, degp_ref, b1_ref, w2_ref, y2_ref):
    dis = _dis(degp_ref[...])
    agg = p_ref[0] + p_ref[1] + y1_ref[...]
    h = jnp.maximum(agg * dis + b1_ref[...], 0.0)
    y2_ref[...] = jnp.dot(h, w2_ref[...], preferred_element_type=jnp.float32) * dis


def _k_final(p_ref, y2_ref, degp_ref, b2_ref, o_ref):
    dis = _dis(degp_ref[...])
    o_ref[...] = (p_ref[0] + p_ref[1] + y2_ref[...]) * dis + b2_ref[...]


_row_spec = pl.BlockSpec((RB, F), lambda i: (i, 0))
_w_spec = pl.BlockSpec((F, F), lambda i: (0, 0))
_degp_spec = pl.BlockSpec((RB, 2), lambda i: (i, 0))
_p_spec = pl.BlockSpec((2, RB, F), lambda i: (0, i, 0))
_b_spec = pl.BlockSpec((1, F), lambda i: (0, 0))
_out_sds = jax.ShapeDtypeStruct((N_PAD, F), jnp.float32)


def _scale_mm(x, W, degp_r):
    return pl.pallas_call(
        _k_scale_mm, grid=(NBLK,),
        in_specs=[_row_spec, _w_spec, _degp_spec],
        out_specs=_row_spec, out_shape=_out_sds,
    )(x, W, degp_r)


def _layer(p, y1, degp_r, b1, W2):
    return pl.pallas_call(
        _k_layer, grid=(NBLK,),
        in_specs=[_p_spec, _row_spec, _degp_spec, _b_spec, _w_spec],
        out_specs=_row_spec, out_shape=_out_sds,
    )(p, y1, degp_r, b1, W2)


def _final(p, y2, degp_r, b2):
    return pl.pallas_call(
        _k_final, grid=(NBLK,),
        in_specs=[_p_spec, _row_spec, _degp_spec, _b_spec],
        out_specs=_row_spec, out_shape=_out_sds,
    )(p, y2, degp_r, b2)


def kernel(x, edge_index, W1, b1, W2, b2):
    n = x.shape[0]
    e = edge_index.shape[1]
    src = edge_index[0].astype(jnp.int32)
    dst = edge_index[1].astype(jnp.int32)
    pad_idx = jnp.full((NE_PAD - e,), n, jnp.int32)
    src_t = jnp.concatenate([src, pad_idx]).reshape(NW, NB, B)
    dst_t = jnp.concatenate([dst, pad_idx]).reshape(NW, NB, B)
    pad_idx2 = jnp.full((NE_PAD2 - e,), n, jnp.int32)
    src_h = jnp.concatenate([src, pad_idx2]).reshape(2, NW, NBH, B)
    dst_h = jnp.concatenate([dst, pad_idx2]).reshape(2, NW, NBH, B)
    x_p = jnp.pad(x, ((0, N_PAD - n), (0, 0)))
    zeros3d = jnp.zeros((NC, N_PAD, F), jnp.float32)
    zeros1d = jnp.zeros((N_PAD,), jnp.float32)

    def agg(y):
        p_half = _agg_kernel(y, src_h[0], dst_h[0], zeros3d)
        return _agg_kernel(y, src_h[1], dst_h[1], p_half)

    degp = _deg_kernel(dst_t, zeros1d)
    degp_r = degp.T
    y1 = _scale_mm(x_p, W1, degp_r)
    p1 = agg(y1)
    y2 = _layer(p1, y1, degp_r, b1.reshape(1, F), W2)
    p2 = agg(y2)
    out = _final(p2, y2, degp_r, b2.reshape(1, F))
    return out[:n]


# speedup vs baseline: 1.3269x; 1.3269x over previous
import functools

import jax
import jax.numpy as jnp
from jax import lax
from jax.experimental import pallas as pl
from jax.experimental.pallas import tpu as pltpu
from jax.experimental.pallas import tpu_sc as plsc

F = 128

NC = 2
NS = 16
NW = NC * NS
B = 128
NB = 79
NE_PAD = NW * NB * B
RB = 1024
N_PAD = 10240
NBLK = N_PAD // RB
RPT = N_PAD // NS

_mesh = plsc.VectorSubcoreMesh(core_axis_name="c", subcore_axis_name="s")


@functools.partial(
    pl.kernel,
    mesh=_mesh,
    out_type=jax.ShapeDtypeStruct((NC, N_PAD), jnp.float32),
    scratch_types=[
        pltpu.VMEM((NB, B), jnp.int32),
        pltpu.VMEM((B,), jnp.float32),
        pltpu.VMEM_SHARED((N_PAD,), jnp.float32),
    ],
)
def _deg_kernel(dst_hbm, zeros_hbm, out_hbm, dst_v, ones_v, acc):
    c = lax.axis_index("c")
    s = lax.axis_index("s")
    wid = s * NC + c
    for i in range(B // 16):
        ones_v[pl.ds(i * 16, 16)] = jnp.ones((16,), jnp.float32)
    pltpu.sync_copy(zeros_hbm.at[pl.ds(s * RPT, RPT)], acc.at[pl.ds(s * RPT, RPT)])
    pltpu.sync_copy(dst_hbm.at[wid], dst_v)
    plsc.subcore_barrier()

    def body(j, carry):
        pltpu.sync_copy(ones_v, acc.at[dst_v.at[j]], add=True)
        return carry

    lax.fori_loop(0, NB, body, 0)
    plsc.subcore_barrier()
    pltpu.sync_copy(acc.at[pl.ds(s * RPT, RPT)], out_hbm.at[c, pl.ds(s * RPT, RPT)])


@functools.partial(
    pl.kernel,
    mesh=_mesh,
    out_type=jax.ShapeDtypeStruct((NC, N_PAD, F), jnp.float32),
    scratch_types=[
        pltpu.VMEM((NB, B), jnp.int32),
        pltpu.VMEM((NB, B), jnp.int32),
        pltpu.VMEM((B, F), jnp.float32),
        pltpu.VMEM_SHARED((N_PAD, F), jnp.float32),
        pltpu.SemaphoreType.DMA,
    ],
)
def _agg_kernel(y_hbm, src_hbm, dst_hbm, zeros_hbm, out_hbm,
                src_v, dst_v, rows_v, acc, sem):
    c = lax.axis_index("c")
    s = lax.axis_index("s")
    wid = s * NC + c
    pltpu.sync_copy(zeros_hbm.at[pl.ds(s * RPT, RPT)], acc.at[pl.ds(s * RPT, RPT)])
    pltpu.sync_copy(src_hbm.at[wid], src_v)
    pltpu.sync_copy(dst_hbm.at[wid], dst_v)
    plsc.subcore_barrier()

    def body(j, carry):
        pltpu.async_copy(y_hbm.at[src_v.at[j]], rows_v, sem).wait()
        pltpu.sync_copy(rows_v, acc.at[dst_v.at[j]], add=True)
        return carry

    lax.fori_loop(0, NB, body, 0)
    plsc.subcore_barrier()
    pltpu.sync_copy(acc.at[pl.ds(s * RPT, RPT)], out_hbm.at[c, pl.ds(s * RPT, RPT)])


def _dis(degp_blk):
    deg = degp_blk[:, 0:1] + degp_blk[:, 1:2] + 1.0
    return lax.rsqrt(deg)


def _k_scale_mm(x_ref, w_ref, degp_ref, y_ref):
    xw = jnp.dot(x_ref[...], w_ref[...], preferred_element_type=jnp.float32)
    y_ref[...] = xw * _dis(degp_ref[...])


def _k_layer(p_ref, y1_ref, degp_ref, b1_ref, w2_ref, y2_ref):
    dis = _dis(degp_ref[...])
    agg = p_ref[0] + p_ref[1] + y1_ref[...]
    h = jnp.maximum(agg * dis + b1_ref[...], 0.0)
    y2_ref[...] = jnp.dot(h, w2_ref[...], preferred_element_type=jnp.float32) * dis


def _k_final(p_ref, y2_ref, degp_ref, b2_ref, o_ref):
    dis = _dis(degp_ref[...])
    o_ref[...] = (p_ref[0] + p_ref[1] + y2_ref[...]) * dis + b2_ref[...]


_row_spec = pl.BlockSpec((RB, F), lambda i: (i, 0))
_w_spec = pl.BlockSpec((F, F), lambda i: (0, 0))
_degp_spec = pl.BlockSpec((RB, 2), lambda i: (i, 0))
_p_spec = pl.BlockSpec((2, RB, F), lambda i: (0, i, 0))
_b_spec = pl.BlockSpec((1, F), lambda i: (0, 0))
_out_sds = jax.ShapeDtypeStruct((N_PAD, F), jnp.float32)


def _scale_mm(x, W, degp_r):
    return pl.pallas_call(
        _k_scale_mm, grid=(NBLK,),
        in_specs=[_row_spec, _w_spec, _degp_spec],
        out_specs=_row_spec, out_shape=_out_sds,
    )(x, W, degp_r)


def _layer(p, y1, degp_r, b1, W2):
    return pl.pallas_call(
        _k_layer, grid=(NBLK,),
        in_specs=[_p_spec, _row_spec, _degp_spec, _b_spec, _w_spec],
        out_specs=_row_spec, out_shape=_out_sds,
    )(p, y1, degp_r, b1, W2)


def _final(p, y2, degp_r, b2):
    return pl.pallas_call(
        _k_final, grid=(NBLK,),
        in_specs=[_p_spec, _row_spec, _degp_spec, _b_spec],
        out_specs=_row_spec, out_shape=_out_sds,
    )(p, y2, degp_r, b2)


def kernel(x, edge_index, W1, b1, W2, b2):
    n = x.shape[0]
    e = edge_index.shape[1]
    src = edge_index[0].astype(jnp.int32)
    dst = edge_index[1].astype(jnp.int32)
    pad_idx = jnp.full((NE_PAD - e,), n, jnp.int32)
    src_t = jnp.concatenate([src, pad_idx]).reshape(NW, NB, B)
    dst_t = jnp.concatenate([dst, pad_idx]).reshape(NW, NB, B)
    x_p = jnp.pad(x, ((0, N_PAD - n), (0, 0)))
    zeros2d = jnp.zeros((N_PAD, F), jnp.float32)
    zeros1d = jnp.zeros((N_PAD,), jnp.float32)

    degp = _deg_kernel(dst_t, zeros1d)
    degp_r = degp.T
    y1 = _scale_mm(x_p, W1, degp_r)
    p1 = _agg_kernel(y1, src_t, dst_t, zeros2d)
    y2 = _layer(p1, y1, degp_r, b1.reshape(1, F), W2)
    p2 = _agg_kernel(y2, src_t, dst_t, zeros2d)
    out = _final(p2, y2, degp_r, b2.reshape(1, F))
    return out[:n]
